# baseline (speedup 1.0000x reference)
import functools

import jax
import jax.numpy as jnp
from jax import lax
from jax.experimental import pallas as pl
from jax.experimental.pallas import tpu as pltpu
from jax.experimental.pallas import tpu_sc as plsc

_EMBED_DIM = 64
_GROUP = 128


def _make_gather(num_groups: int):
  info = plsc.get_sparse_core_info()
  nc, ns = info.num_cores, info.num_subcores
  nw = nc * ns
  assert num_groups % nw == 0
  gpw = num_groups // nw

  mesh = plsc.VectorSubcoreMesh(core_axis_name="c", subcore_axis_name="s")

  @functools.partial(
      pl.kernel,
      mesh=mesh,
      out_type=jax.ShapeDtypeStruct((num_groups * _GROUP, _EMBED_DIM),
                                    jnp.float32),
      scratch_types=[
          pltpu.VMEM((gpw, _GROUP), jnp.int32),
          pltpu.VMEM((_GROUP, _EMBED_DIM), jnp.float32),
          pltpu.SemaphoreType.DMA,
      ],
      compiler_params=pltpu.CompilerParams(use_tc_tiling_on_sc=False),
  )
  def gather_kernel(idx_hbm, table_hbm, out_hbm, idx_v, rows_v, gsem):
    wid = lax.axis_index("s") * nc + lax.axis_index("c")
    g0 = wid * gpw
    pltpu.sync_copy(idx_hbm.at[wid], idx_v)

    def step(j, carry):
      pltpu.async_copy(table_hbm.at[idx_v.at[j]], rows_v, gsem).wait()
      pltpu.sync_copy(rows_v, out_hbm.at[pl.ds((g0 + j) * _GROUP, _GROUP)])
      return carry

    lax.fori_loop(0, gpw, step, 0)

  return gather_kernel


def kernel(input_ids, word_embeddings):
  batch, seq = input_ids.shape
  vocab, dim = word_embeddings.shape
  n = batch * seq
  num_groups = n // _GROUP
  nw = 32
  idx = input_ids.reshape(nw, num_groups // nw, _GROUP).astype(jnp.int32)
  out = _make_gather(num_groups)(idx, word_embeddings)
  return out.reshape(batch, seq, dim)


# speedup vs baseline: 1.0479x; 1.0479x over previous
import functools

import jax
import jax.numpy as jnp
from jax import lax
from jax.experimental import pallas as pl
from jax.experimental.pallas import tpu as pltpu
from jax.experimental.pallas import tpu_sc as plsc

_EMBED_DIM = 64
_GROUP = 128
_KPM = 5
_MROWS = _GROUP * _KPM


def _make_gather(num_groups: int):
  info = plsc.get_sparse_core_info()
  nc, ns = info.num_cores, info.num_subcores
  nw = nc * ns
  assert num_groups % nw == 0
  gpw = num_groups // nw
  assert gpw % _KPM == 0
  nm = gpw // _KPM
  assert nm % 2 == 0 and nm >= 4

  mesh = plsc.VectorSubcoreMesh(core_axis_name="c", subcore_axis_name="s")

  @functools.partial(
      pl.kernel,
      mesh=mesh,
      out_type=jax.ShapeDtypeStruct((num_groups * _GROUP, _EMBED_DIM),
                                    jnp.float32),
      scratch_types=[
          pltpu.VMEM((gpw, _GROUP), jnp.int32),
          pltpu.VMEM((2, _MROWS, _EMBED_DIM), jnp.float32),
          pltpu.SemaphoreType.DMA,
          pltpu.SemaphoreType.DMA,
      ],
      compiler_params=pltpu.CompilerParams(use_tc_tiling_on_sc=False),
  )
  def gather_kernel(idx_hbm, table_hbm, out_hbm, idx_v, rows_v, gsem, wsem):
    wid = lax.axis_index("s") * nc + lax.axis_index("c")
    base = wid * gpw * _GROUP
    pltpu.sync_copy(idx_hbm.at[wid], idx_v)

    def fire_gathers(m, h):
      for b in range(_KPM):
        pltpu.async_copy(
            table_hbm.at[idx_v.at[m * _KPM + b]],
            rows_v.at[h].at[pl.ds(b * _GROUP, _GROUP)], gsem)

    def wait_gathers(m, h):
      for b in range(_KPM):
        pltpu.make_async_copy(
            table_hbm.at[idx_v.at[m * _KPM + b]],
            rows_v.at[h].at[pl.ds(b * _GROUP, _GROUP)], gsem).wait()

    def fire_wb(m, h):
      pltpu.async_copy(
          rows_v.at[h], out_hbm.at[pl.ds(base + m * _MROWS, _MROWS)], wsem)

    def wait_wb(m, h):
      pltpu.make_async_copy(
          rows_v.at[h], out_hbm.at[pl.ds(base + m * _MROWS, _MROWS)],
          wsem).wait()

    fire_gathers(0, 0)
    fire_gathers(1, 1)

    def step(k, carry):
      m = 2 * k
      for h in (0, 1):
        wait_gathers(m + h, h)
        fire_wb(m + h, h)
        wait_wb(m + h, h)
        fire_gathers(m + h + 2, h)
      return carry

    lax.fori_loop(0, (nm - 2) // 2, step, 0)

    for h in (0, 1):
      m = nm - 2 + h
      wait_gathers(m, h)
      fire_wb(m, h)
      wait_wb(m, h)

  return gather_kernel


def kernel(input_ids, word_embeddings):
  batch, seq = input_ids.shape
  vocab, dim = word_embeddings.shape
  n = batch * seq
  num_groups = n // _GROUP
  nw = 32
  idx = input_ids.reshape(nw, num_groups // nw, _GROUP).astype(jnp.int32)
  out = _make_gather(num_groups)(idx, word_embeddings)
  return out.reshape(batch, seq, dim)


# speedup vs baseline: 1.0568x; 1.0085x over previous
import functools

import jax
import jax.numpy as jnp
from jax import lax
from jax.experimental import pallas as pl
from jax.experimental.pallas import tpu as pltpu
from jax.experimental.pallas import tpu_sc as plsc

_EMBED_DIM = 64
_GROUP = 128
_KPM = 5
_MROWS = _GROUP * _KPM


def _make_gather(num_groups: int):
  info = plsc.get_sparse_core_info()
  nc, ns = info.num_cores, info.num_subcores
  nw = nc * ns
  assert num_groups % nw == 0
  gpw = num_groups // nw
  assert gpw % _KPM == 0
  nm = gpw // _KPM
  assert nm % 2 == 0 and nm >= 4

  mesh = plsc.VectorSubcoreMesh(core_axis_name="c", subcore_axis_name="s")

  @functools.partial(
      pl.kernel,
      mesh=mesh,
      out_type=jax.ShapeDtypeStruct((num_groups * _GROUP, _EMBED_DIM),
                                    jnp.float32),
      scratch_types=[
          pltpu.VMEM((gpw, _GROUP), jnp.int32),
          pltpu.VMEM((2, _MROWS, _EMBED_DIM), jnp.float32),
          pltpu.SemaphoreType.DMA,
          pltpu.SemaphoreType.DMA,
      ],
      compiler_params=pltpu.CompilerParams(use_tc_tiling_on_sc=False),
  )
  def gather_kernel(idx_hbm, table_hbm, out_hbm, idx_v, rows_v, gsem, wsem):
    wid = lax.axis_index("s") * nc + lax.axis_index("c")
    base = wid * gpw * _GROUP
    pltpu.sync_copy(idx_hbm.at[wid], idx_v)

    def fire_gathers(m, h):
      for b in range(_KPM):
        pltpu.async_copy(
            table_hbm.at[idx_v.at[m * _KPM + b]],
            rows_v.at[h].at[pl.ds(b * _GROUP, _GROUP)], gsem)

    def wait_gathers(m, h):
      for b in range(_KPM):
        pltpu.make_async_copy(
            table_hbm.at[idx_v.at[m * _KPM + b]],
            rows_v.at[h].at[pl.ds(b * _GROUP, _GROUP)], gsem).wait()

    def fire_wb(m, h):
      pltpu.async_copy(
          rows_v.at[h], out_hbm.at[pl.ds(base + m * _MROWS, _MROWS)], wsem)

    def wait_wb(m, h):
      pltpu.make_async_copy(
          rows_v.at[h], out_hbm.at[pl.ds(base + m * _MROWS, _MROWS)],
          wsem).wait()

    fire_gathers(0, 0)
    fire_gathers(1, 1)

    def step(k, carry):
      m = 2 * k
      for h in (0, 1):
        wait_gathers(m + h, h)
        fire_wb(m + h, h)
        wait_wb(m + h, h)
        fire_gathers(m + h + 2, h)
      return carry

    lax.fori_loop(0, (nm - 2) // 2, step, 0)

    for h in (0, 1):
      m = nm - 2 + h
      wait_gathers(m, h)
      fire_wb(m, h)
      wait_wb(m, h)

  return gather_kernel


def kernel(input_ids, word_embeddings):
  batch, seq = input_ids.shape
  vocab, dim = word_embeddings.shape
  n = batch * seq
  num_groups = n // _GROUP
  nw = 32
  idx = input_ids.T.reshape(nw, num_groups // nw, _GROUP)
  out = _make_gather(num_groups)(idx, word_embeddings)
  return out.reshape(seq, batch, dim).transpose(1, 0, 2)
